# Initial kernel scaffold; baseline (speedup 1.0000x reference)
import functools

import jax
import jax.numpy as jnp
from jax import lax
from jax.experimental import pallas as pl
from jax.experimental.pallas import tpu as pltpu
from jax.experimental.pallas import tpu_sc as plsc

_FIELD = 26
_D = 16
_BATCH = 16384
_VOCAB = 40000
_NC = 2
_NS = 16
_NW = _NC * _NS
_RPW = _BATCH // _NW
_CHUNK = 64
_NCHUNK = _RPW // _CHUNK
_IPC = _CHUNK * _FIELD
_NDMA = _IPC // 128
_NGRP = _CHUNK // 16
_IDX_ROWS_PER_W = _RPW * _FIELD // 128


def _fm_body(emb, fcf, idx, out, idx_v, rows_v, fc_v, out_v, gsem):
    wid = lax.axis_index("s") * _NC + lax.axis_index("c")
    idx_base = wid * _IDX_ROWS_PER_W
    out_base = wid * _RPW
    lane = lax.iota(jnp.int32, 16)

    def chunk_body(c, carry):
        pltpu.sync_copy(idx.at[pl.ds(idx_base + c * _NDMA, _NDMA)], idx_v)
        copies = []
        for j in range(_NDMA):
            copies.append(
                pltpu.async_copy(emb.at[idx_v.at[j]],
                                 rows_v.at[pl.ds(j * 128, 128)], gsem))
            copies.append(
                pltpu.async_copy(fcf.at[idx_v.at[j]],
                                 fc_v.at[pl.ds(j * 128, 128)], gsem))
        for h in copies:
            h.wait()

        def grp_body(g, carry2):
            row0 = lane * _FIELD + g * (16 * _FIELD)
            rows = [row0 + f for f in range(_FIELD)]
            facc = plsc.load_gather(fc_v, [rows[0]])
            for f in range(1, _FIELD):
                facc = facc + plsc.load_gather(fc_v, [rows[f]])
            acc = jnp.zeros((16,), jnp.float32)
            for d in range(_D):
                col = jnp.full((16,), d, jnp.int32)
                s = jnp.zeros((16,), jnp.float32)
                ss = jnp.zeros((16,), jnp.float32)
                for f in range(_FIELD):
                    v = plsc.load_gather(rows_v, [rows[f], col])
                    s = s + v
                    ss = ss + v * v
                acc = acc + (s * s - ss)
            out_v[pl.ds(g * 16, 16)] = facc + 0.5 * acc
            return carry2

        lax.fori_loop(0, _NGRP, grp_body, 0)
        pltpu.sync_copy(out_v, out.at[pl.ds(out_base + c * _CHUNK, _CHUNK)])
        return carry

    lax.fori_loop(0, _NCHUNK, chunk_body, 0)


_fm = functools.partial(
    pl.kernel,
    out_type=jax.ShapeDtypeStruct((_BATCH,), jnp.float32),
    mesh=plsc.VectorSubcoreMesh(
        core_axis_name="c", subcore_axis_name="s",
        num_cores=_NC, num_subcores=_NS),
    scratch_types=[
        pltpu.VMEM((_NDMA, 128), jnp.int32),
        pltpu.VMEM((_IPC, _D), jnp.float32),
        pltpu.VMEM((_IPC,), jnp.float32),
        pltpu.VMEM((_CHUNK,), jnp.float32),
        pltpu.SemaphoreType.DMA,
    ],
)(_fm_body)


@jax.jit
def kernel(x, emb_table, fc_table, bias):
    offs = jnp.arange(_FIELD, dtype=jnp.int32) * _VOCAB
    idx = (x + offs[None, :]).reshape(_BATCH * _FIELD // 128, 128)
    out = _fm(emb_table, fc_table.reshape(-1), idx)
    return out + bias[0]


# speedup vs baseline: 1.1300x; 1.1300x over previous
import functools

import jax
import jax.numpy as jnp
from jax import lax
from jax.experimental import pallas as pl
from jax.experimental.pallas import tpu as pltpu
from jax.experimental.pallas import tpu_sc as plsc

_FIELD = 26
_D = 16
_BATCH = 16384
_VOCAB = 40000
_NC = 2
_NS = 16
_NW = _NC * _NS
_RPW = _BATCH // _NW
_CHUNK = 64
_NCHUNK = _RPW // _CHUNK
_IPC = _CHUNK * _FIELD
_NDMA = _IPC // 128
_NGRP = _CHUNK // 16
_IDX_ROWS_PER_W = _RPW * _FIELD // 128


def _fm_body(emb, fcf, idx, out, idx_v, rows_v, fc_v, out_v, gsem):
    wid = lax.axis_index("s") * _NC + lax.axis_index("c")
    chunk_base = wid * _NCHUNK
    out_base = wid * _RPW
    lane = lax.iota(jnp.int32, 16)

    def chunk_body(c, carry):
        pltpu.sync_copy(idx.at[chunk_base + c], idx_v)
        copies = []
        for j in range(_NDMA):
            copies.append(
                pltpu.async_copy(emb.at[idx_v.at[j]],
                                 rows_v.at[pl.ds(j * 128, 128)], gsem))
            copies.append(
                pltpu.async_copy(fcf.at[idx_v.at[j]],
                                 fc_v.at[pl.ds(j * 128, 128)], gsem))
        for h in copies:
            h.wait()

        def grp_body(g, carry2):
            row0 = lane * _FIELD + g * (16 * _FIELD)
            rows = [row0 + f for f in range(_FIELD)]
            facc = plsc.load_gather(fc_v, [rows[0]])
            for f in range(1, _FIELD):
                facc = facc + plsc.load_gather(fc_v, [rows[f]])
            acc = jnp.zeros((16,), jnp.float32)
            for d in range(_D):
                col = jnp.full((16,), d, jnp.int32)
                s = jnp.zeros((16,), jnp.float32)
                ss = jnp.zeros((16,), jnp.float32)
                for f in range(_FIELD):
                    v = plsc.load_gather(rows_v, [rows[f], col])
                    s = s + v
                    ss = ss + v * v
                acc = acc + (s * s - ss)
            out_v[pl.ds(g * 16, 16)] = facc + 0.5 * acc
            return carry2

        lax.fori_loop(0, _NGRP, grp_body, 0)
        pltpu.sync_copy(out_v, out.at[pl.ds(out_base + c * _CHUNK, _CHUNK)])
        return carry

    lax.fori_loop(0, _NCHUNK, chunk_body, 0)


_fm = functools.partial(
    pl.kernel,
    out_type=jax.ShapeDtypeStruct((_BATCH,), jnp.float32),
    mesh=plsc.VectorSubcoreMesh(
        core_axis_name="c", subcore_axis_name="s",
        num_cores=_NC, num_subcores=_NS),
    compiler_params=pltpu.CompilerParams(
        needs_layout_passes=False, use_tc_tiling_on_sc=False),
    scratch_types=[
        pltpu.VMEM((_NDMA, 128), jnp.int32),
        pltpu.VMEM((_IPC, _D), jnp.float32),
        pltpu.VMEM((_IPC,), jnp.float32),
        pltpu.VMEM((_CHUNK,), jnp.float32),
        pltpu.SemaphoreType.DMA,
    ],
)(_fm_body)


@jax.jit
def kernel(x, emb_table, fc_table, bias):
    offs = jnp.arange(_FIELD, dtype=jnp.int32) * _VOCAB
    idx = (x + offs[None, :]).reshape(_NW * _NCHUNK, _NDMA, 128)
    out = _fm(emb_table, fc_table.reshape(-1), idx)
    return out + bias[0]


# speedup vs baseline: 1.2263x; 1.0852x over previous
import functools

import jax
import jax.numpy as jnp
from jax import lax
from jax.experimental import pallas as pl
from jax.experimental.pallas import tpu as pltpu
from jax.experimental.pallas import tpu_sc as plsc

_FIELD = 26
_D = 16
_BATCH = 16384
_VOCAB = 40000
_NC = 2
_NS = 16
_NW = _NC * _NS
_RPW = _BATCH // _NW
_CHUNK = 64
_NCHUNK = _RPW // _CHUNK
_IPC = _CHUNK * _FIELD


def _fm_body(emb, fcf, idx, out, idx_v, rows_v, fc_v, out_v, gsem):
    wid = lax.axis_index("s") * _NC + lax.axis_index("c")
    row_base = wid * _RPW
    lane = lax.iota(jnp.int32, 16)

    pltpu.sync_copy(idx.at[:, pl.ds(row_base, _RPW)], idx_v)

    def chunk_body(c, carry):
        copies = []
        for f in range(_FIELD):
            iv = idx_v.at[f, pl.ds(c * _CHUNK, _CHUNK)]
            copies.append(
                pltpu.async_copy(emb.at[iv],
                                 rows_v.at[pl.ds(f * _CHUNK, _CHUNK)], gsem))
            copies.append(
                pltpu.async_copy(fcf.at[iv],
                                 fc_v.at[pl.ds(f * _CHUNK, _CHUNK)], gsem))
        for h in copies:
            h.wait()

        def grp_body(g, carry2):
            row0 = lane + g * 16
            rows = [row0 + f * _CHUNK for f in range(_FIELD)]
            facc = plsc.load_gather(fc_v, [rows[0]])
            for f in range(1, _FIELD):
                facc = facc + plsc.load_gather(fc_v, [rows[f]])
            acc = jnp.zeros((16,), jnp.float32)
            for d in range(_D):
                col = jnp.full((16,), d, jnp.int32)
                s = jnp.zeros((16,), jnp.float32)
                ss = jnp.zeros((16,), jnp.float32)
                for f in range(_FIELD):
                    v = plsc.load_gather(rows_v, [rows[f], col])
                    s = s + v
                    ss = ss + v * v
                acc = acc + (s * s - ss)
            out_v[pl.ds(g * 16, 16)] = facc + 0.5 * acc
            return carry2

        lax.fori_loop(0, _CHUNK // 16, grp_body, 0)
        pltpu.sync_copy(out_v, out.at[pl.ds(row_base + c * _CHUNK, _CHUNK)])
        return carry

    lax.fori_loop(0, _NCHUNK, chunk_body, 0)


_fm = functools.partial(
    pl.kernel,
    out_type=jax.ShapeDtypeStruct((_BATCH,), jnp.float32),
    mesh=plsc.VectorSubcoreMesh(
        core_axis_name="c", subcore_axis_name="s",
        num_cores=_NC, num_subcores=_NS),
    compiler_params=pltpu.CompilerParams(
        needs_layout_passes=False, use_tc_tiling_on_sc=False),
    scratch_types=[
        pltpu.VMEM((_FIELD, _RPW), jnp.int32),
        pltpu.VMEM((_IPC, _D), jnp.float32),
        pltpu.VMEM((_IPC,), jnp.float32),
        pltpu.VMEM((_CHUNK,), jnp.float32),
        pltpu.SemaphoreType.DMA,
    ],
)(_fm_body)


@jax.jit
def kernel(x, emb_table, fc_table, bias):
    offs = jnp.arange(_FIELD, dtype=jnp.int32) * _VOCAB
    idx_t = x.T + offs[:, None]
    out = _fm(emb_table, fc_table.reshape(-1), idx_t)
    return out + bias[0]
